# baseline (speedup 1.0000x reference)
import functools

import jax
import jax.numpy as jnp
from jax import lax
from jax.experimental import pallas as pl
from jax.experimental.pallas import tpu as pltpu
from jax.experimental.pallas import tpu_sc as plsc

N = 10000
D = 128
E = 320000
NR = 7

REL_SRC_T = (0, 2, 0, 3, 3, 3, 1)
REL_DST_T = (2, 0, 3, 0, 3, 1, 3)

NW = 32
EPT = E // NW
CHUNK = 64
NCH = 160
EPT_PAD = NCH * CHUNK
ACC_ROWS = 10240
RPT = ACC_ROWS // 16
ZCH = 8


def _sc_segment_sums(x_stack, src_idx, dst_idx):
    mesh = plsc.VectorSubcoreMesh(core_axis_name="c", subcore_axis_name="s")

    @functools.partial(
        pl.kernel,
        mesh=mesh,
        out_type=(
            jax.ShapeDtypeStruct((NR, 2, ACC_ROWS, D), jnp.float32),
            jax.ShapeDtypeStruct((NR, 2, ACC_ROWS // 8, D), jnp.float32),
        ),
        scratch_types=[
            pltpu.VMEM_SHARED((ACC_ROWS, D), jnp.float32),
            pltpu.VMEM((NCH // 2, CHUNK), jnp.int32),
            pltpu.VMEM((NCH // 2, CHUNK), jnp.int32),
            pltpu.VMEM((CHUNK, D), jnp.float32),
            pltpu.VMEM((CHUNK, D), jnp.float32),
            pltpu.VMEM((CHUNK, D), jnp.float32),
            pltpu.SemaphoreType.DMA,
            pltpu.SemaphoreType.DMA,
            pltpu.SemaphoreType.DMA,
        ],
    )
    def k(x_hbm, src_hbm, dst_hbm, acc_out, cnt_out,
          sh, src_v, dst_v, rows_a, rows_b, rows_c, sem_a, sem_b, sem_c):
        cid = lax.axis_index("c")
        sid = lax.axis_index("s")
        wid = cid * 16 + sid
        base = sid * RPT
        H = NCH // 2

        z16 = jnp.zeros((16,), jnp.float32)
        o16 = jnp.ones((16,), jnp.float32)

        def fill(buf, v16):
            def fb(i, cc):
                for kk in range(D // 16):
                    buf[i, pl.ds(kk * 16, 16)] = v16
                return cc

            lax.fori_loop(0, CHUNK, fb, 0)

        def zero_own_rows(zsrc):
            def zero_body(i, cc):
                pltpu.sync_copy(zsrc, sh.at[pl.ds(base + i * CHUNK, CHUNK)])
                return cc

            lax.fori_loop(0, RPT // CHUNK, zero_body, 0)

        def rel_body(r, c):
            fill(rows_b, z16)
            zero_own_rows(rows_b)
            plsc.subcore_barrier()
            for half in range(2):
                off = half * H
                pltpu.sync_copy(src_hbm.at[r, wid, pl.ds(off, H)], src_v)
                pltpu.sync_copy(dst_hbm.at[r, wid, pl.ds(off, H)], dst_v)
                bufs = (rows_a, rows_b, rows_c)
                sems = (sem_a, sem_b, sem_c)
                pltpu.async_copy(x_hbm.at[src_v.at[0]], rows_a, sem_a)
                pltpu.async_copy(x_hbm.at[src_v.at[1]], rows_b, sem_b)

                def pipe(g, cc):
                    for u in range(3):
                        j = 3 * g + u
                        pltpu.async_copy(
                            x_hbm.at[src_v.at[j + 2]], bufs[(u + 2) % 3],
                            sems[(u + 2) % 3])
                        pltpu.make_async_copy(
                            x_hbm.at[src_v.at[j]], bufs[u], sems[u]).wait()
                        pltpu.sync_copy(bufs[u], sh.at[dst_v.at[j]], add=True)
                    return cc

                lax.fori_loop(0, H // 3, pipe, 0)
                for (j, u) in ((H - 2, 0), (H - 1, 1)):
                    pltpu.make_async_copy(
                        x_hbm.at[src_v.at[j]], bufs[u], sems[u]).wait()
                    pltpu.sync_copy(bufs[u], sh.at[dst_v.at[j]], add=True)
            plsc.subcore_barrier()
            pltpu.sync_copy(sh.at[pl.ds(base, RPT)],
                            acc_out.at[r, cid, pl.ds(base, RPT)])

            fill(rows_b, z16)
            zero_own_rows(rows_b)
            fill(rows_a, o16)
            plsc.subcore_barrier()
            for half in (1, 0):
                if half == 0:
                    pltpu.sync_copy(dst_hbm.at[r, wid, pl.ds(0, H)], dst_v)

                def cnt_blk(j, cc):
                    for u in range(16):
                        pltpu.async_copy(
                            rows_a, sh.at[dst_v.at[16 * j + u]], sem_a, add=True)
                    for u in range(16):
                        pltpu.make_async_copy(
                            rows_a, sh.at[dst_v.at[16 * j + u]], sem_a).wait()
                    return cc

                lax.fori_loop(0, H // 16, cnt_blk, 0)
            plsc.subcore_barrier()

            def pack_blk(b, cc):
                pltpu.sync_copy(sh.at[pl.ds(base + b * 64, 64)], rows_b)

                def pack_row(jj, c2):
                    for m in range(8):
                        rows_c[jj, pl.ds(m * 16, 16)] = \
                            rows_b[jj * 8 + m, pl.ds(0, 16)]
                    return c2

                lax.fori_loop(0, 8, pack_row, 0)
                pltpu.sync_copy(
                    rows_c.at[pl.ds(0, 8)],
                    cnt_out.at[r, cid, pl.ds(sid * (RPT // 8) + b * 8, 8)])
                return cc

            lax.fori_loop(0, RPT // 64, pack_blk, 0)
            return c

        lax.fori_loop(0, NR, rel_body, 0)

    return k(x_stack, src_idx, dst_idx)


def _tc_combine(xs, W_roots, bs, W_rels, acc, cnt):
    B = 1000
    grid = (N // B,)

    def body(xa, xf, xi, xp, wr, bb, wl, acc_r, cnt_r, oa, of, oi, op):
        accv = acc_r[...]
        cntv = cnt_r[...]
        a = accv[:, 0] + accv[:, 1]
        ccount = cntv[:, 0, :, 0] + cntv[:, 1, :, 0]
        inv = 1.0 / jnp.maximum(ccount, 1.0)
        agg = a * inv[:, :, None]
        wrv = wr[...]
        bv = bb[...]
        wlv = wl[...]
        xv = [xa[...], xf[...], xi[...], xp[...]]
        dn = (((1,), (1,)), ((), ()))
        outs = [
            lax.dot_general(xv[t], wrv[t], dn,
                            preferred_element_type=jnp.float32) + bv[t][None, :]
            for t in range(4)
        ]
        for r in range(NR):
            outs[REL_DST_T[r]] = outs[REL_DST_T[r]] + lax.dot_general(
                agg[r], wlv[r], dn, preferred_element_type=jnp.float32)
        oa[...] = outs[0]
        of[...] = outs[1]
        oi[...] = outs[2]
        op[...] = outs[3]

    row_spec = pl.BlockSpec((B, D), lambda i: (i, 0))
    return pl.pallas_call(
        body,
        grid=grid,
        in_specs=[
            row_spec, row_spec, row_spec, row_spec,
            pl.BlockSpec((4, D, D), lambda i: (0, 0, 0)),
            pl.BlockSpec((4, D), lambda i: (0, 0)),
            pl.BlockSpec((NR, D, D), lambda i: (0, 0, 0)),
            pl.BlockSpec((NR, 2, B, D), lambda i: (0, 0, i, 0)),
            pl.BlockSpec((NR, 2, B, 1), lambda i: (0, 0, i, 0)),
        ],
        out_specs=[row_spec, row_spec, row_spec, row_spec],
        out_shape=[jax.ShapeDtypeStruct((N, D), jnp.float32)] * 4,
    )(*xs, W_roots, bs, W_rels, acc, cnt)


def kernel(x_author, W_root_author, b_root_author,
           x_field_of_study, W_root_field_of_study, b_root_field_of_study,
           x_institution, W_root_institution, b_root_institution,
           x_paper, W_root_paper, b_root_paper,
           W_rel_author_affiliated_with_institution, ei_author_affiliated_with_institution,
           W_rel_institution_to_author, ei_institution_to_author,
           W_rel_author_writes_paper, ei_author_writes_paper,
           W_rel_paper_to_author, ei_paper_to_author,
           W_rel_paper_cites_paper, ei_paper_cites_paper,
           W_rel_paper_has_topic_field_of_study, ei_paper_has_topic_field_of_study,
           W_rel_field_of_study_to_paper, ei_field_of_study_to_paper):
    xs = [x_author, x_field_of_study, x_institution, x_paper]
    eis = [ei_author_affiliated_with_institution, ei_institution_to_author,
           ei_author_writes_paper, ei_paper_to_author, ei_paper_cites_paper,
           ei_paper_has_topic_field_of_study, ei_field_of_study_to_paper]
    W_rels = [W_rel_author_affiliated_with_institution, W_rel_institution_to_author,
              W_rel_author_writes_paper, W_rel_paper_to_author, W_rel_paper_cites_paper,
              W_rel_paper_has_topic_field_of_study, W_rel_field_of_study_to_paper]

    x_stack = jnp.concatenate(xs, axis=0)

    npad = EPT_PAD - EPT
    pad_src = (jnp.arange(npad, dtype=jnp.int32) * 911) % (4 * N)
    pad_dst = N + (jnp.arange(npad, dtype=jnp.int32) % (ACC_ROWS - N))
    pad_src = jnp.broadcast_to(pad_src[None], (NW, npad))
    pad_dst = jnp.broadcast_to(pad_dst[None], (NW, npad))
    src_list, dst_list = [], []
    for r in range(NR):
        ei = eis[r]
        src = (ei[1] + REL_SRC_T[r] * N).astype(jnp.int32).reshape(NW, EPT)
        dst = ei[0].astype(jnp.int32).reshape(NW, EPT)
        src_list.append(jnp.concatenate([src, pad_src], axis=1).reshape(NW, NCH, CHUNK))
        dst_list.append(jnp.concatenate([dst, pad_dst], axis=1).reshape(NW, NCH, CHUNK))
    src_idx = jnp.stack(src_list)
    dst_idx = jnp.stack(dst_list)

    acc, cntp = _sc_segment_sums(x_stack, src_idx, dst_idx)
    cnt = cntp.reshape(NR, 2, ACC_ROWS, 16)[:, :, :, :1]

    W_roots = jnp.stack([W_root_author, W_root_field_of_study,
                         W_root_institution, W_root_paper])
    bs = jnp.stack([b_root_author, b_root_field_of_study,
                    b_root_institution, b_root_paper])
    W_rel_stack = jnp.stack(W_rels)

    outs = _tc_combine(xs, W_roots, bs, W_rel_stack, acc, cnt)
    return tuple(outs)


# speedup vs baseline: 1.0152x; 1.0152x over previous
import functools

import jax
import jax.numpy as jnp
from jax import lax
from jax.experimental import pallas as pl
from jax.experimental.pallas import tpu as pltpu
from jax.experimental.pallas import tpu_sc as plsc

N = 10000
D = 128
E = 320000
NR = 7

REL_SRC_T = (0, 2, 0, 3, 3, 3, 1)
REL_DST_T = (2, 0, 3, 0, 3, 1, 3)

NW = 32
EPT = E // NW
CHUNK = 64
NCH = 160
EPT_PAD = NCH * CHUNK
ACC_ROWS = 10240
RPT = ACC_ROWS // 16
ZCH = 8


def _sc_segment_sums(x_stack, src_idx, dst_idx):
    mesh = plsc.VectorSubcoreMesh(core_axis_name="c", subcore_axis_name="s")

    @functools.partial(
        pl.kernel,
        mesh=mesh,
        out_type=(
            jax.ShapeDtypeStruct((NR, 2, ACC_ROWS, D), jnp.float32),
            jax.ShapeDtypeStruct((NR, 2, ACC_ROWS, D), jnp.float32),
        ),
        scratch_types=[
            pltpu.VMEM_SHARED((ACC_ROWS, D), jnp.float32),
            pltpu.VMEM((NCH // 2, CHUNK), jnp.int32),
            pltpu.VMEM((NCH // 2, CHUNK), jnp.int32),
            pltpu.VMEM((CHUNK, D), jnp.float32),
            pltpu.VMEM((CHUNK, D), jnp.float32),
            pltpu.VMEM((CHUNK, D), jnp.float32),
            pltpu.SemaphoreType.DMA,
            pltpu.SemaphoreType.DMA,
            pltpu.SemaphoreType.DMA,
        ],
    )
    def k(x_hbm, src_hbm, dst_hbm, acc_out, cnt_out,
          sh, src_v, dst_v, rows_a, rows_b, rows_c, sem_a, sem_b, sem_c):
        cid = lax.axis_index("c")
        sid = lax.axis_index("s")
        wid = cid * 16 + sid
        base = sid * RPT
        H = NCH // 2

        z16 = jnp.zeros((16,), jnp.float32)
        o16 = jnp.ones((16,), jnp.float32)

        def fill(buf, v16):
            def fb(i, cc):
                for kk in range(D // 16):
                    buf[i, pl.ds(kk * 16, 16)] = v16
                return cc

            lax.fori_loop(0, CHUNK, fb, 0)

        def zero_own_rows(zsrc):
            def zero_body(i, cc):
                pltpu.sync_copy(zsrc, sh.at[pl.ds(base + i * CHUNK, CHUNK)])
                return cc

            lax.fori_loop(0, RPT // CHUNK, zero_body, 0)

        def rel_body(r, c):
            fill(rows_b, z16)
            zero_own_rows(rows_b)
            plsc.subcore_barrier()
            for half in range(2):
                off = half * H
                pltpu.sync_copy(src_hbm.at[r, wid, pl.ds(off, H)], src_v)
                pltpu.sync_copy(dst_hbm.at[r, wid, pl.ds(off, H)], dst_v)
                bufs = (rows_a, rows_b, rows_c)
                sems = (sem_a, sem_b, sem_c)
                pltpu.async_copy(x_hbm.at[src_v.at[0]], rows_a, sem_a)
                pltpu.async_copy(x_hbm.at[src_v.at[1]], rows_b, sem_b)

                def pipe(g, cc):
                    for u in range(3):
                        j = 3 * g + u
                        pltpu.async_copy(
                            x_hbm.at[src_v.at[j + 2]], bufs[(u + 2) % 3],
                            sems[(u + 2) % 3])
                        pltpu.make_async_copy(
                            x_hbm.at[src_v.at[j]], bufs[u], sems[u]).wait()
                        pltpu.sync_copy(bufs[u], sh.at[dst_v.at[j]], add=True)
                    return cc

                lax.fori_loop(0, H // 3, pipe, 0)
                for (j, u) in ((H - 2, 0), (H - 1, 1)):
                    pltpu.make_async_copy(
                        x_hbm.at[src_v.at[j]], bufs[u], sems[u]).wait()
                    pltpu.sync_copy(bufs[u], sh.at[dst_v.at[j]], add=True)
            plsc.subcore_barrier()
            pltpu.sync_copy(sh.at[pl.ds(base, RPT)],
                            acc_out.at[r, cid, pl.ds(base, RPT)])

            fill(rows_b, z16)
            zero_own_rows(rows_b)
            fill(rows_a, o16)
            plsc.subcore_barrier()
            for half in (1, 0):
                if half == 0:
                    pltpu.sync_copy(dst_hbm.at[r, wid, pl.ds(0, H)], dst_v)

                def cnt_blk(j, cc):
                    for u in range(16):
                        pltpu.async_copy(
                            rows_a, sh.at[dst_v.at[16 * j + u]], sem_a, add=True)
                    for u in range(16):
                        pltpu.make_async_copy(
                            rows_a, sh.at[dst_v.at[16 * j + u]], sem_a).wait()
                    return cc

                lax.fori_loop(0, H // 16, cnt_blk, 0)
            plsc.subcore_barrier()
            pltpu.sync_copy(sh.at[pl.ds(base, RPT)],
                            cnt_out.at[r, cid, pl.ds(base, RPT)])
            return c

        lax.fori_loop(0, NR, rel_body, 0)

    return k(x_stack, src_idx, dst_idx)


def _tc_combine(xs, W_roots, bs, W_rels, acc, cnt):
    B = 1000
    grid = (N // B,)

    def body(xa, xf, xi, xp, wr, bb, wl, acc_r, cnt_r, oa, of, oi, op):
        accv = acc_r[...]
        cntv = cnt_r[...]
        a = accv[:, 0] + accv[:, 1]
        ccount = cntv[:, 0, :, 0] + cntv[:, 1, :, 0]
        inv = 1.0 / jnp.maximum(ccount, 1.0)
        agg = a * inv[:, :, None]
        wrv = wr[...]
        bv = bb[...]
        wlv = wl[...]
        xv = [xa[...], xf[...], xi[...], xp[...]]
        dn = (((1,), (1,)), ((), ()))
        outs = [
            lax.dot_general(xv[t], wrv[t], dn,
                            preferred_element_type=jnp.float32) + bv[t][None, :]
            for t in range(4)
        ]
        for r in range(NR):
            outs[REL_DST_T[r]] = outs[REL_DST_T[r]] + lax.dot_general(
                agg[r], wlv[r], dn, preferred_element_type=jnp.float32)
        oa[...] = outs[0]
        of[...] = outs[1]
        oi[...] = outs[2]
        op[...] = outs[3]

    row_spec = pl.BlockSpec((B, D), lambda i: (i, 0))
    return pl.pallas_call(
        body,
        grid=grid,
        in_specs=[
            row_spec, row_spec, row_spec, row_spec,
            pl.BlockSpec((4, D, D), lambda i: (0, 0, 0)),
            pl.BlockSpec((4, D), lambda i: (0, 0)),
            pl.BlockSpec((NR, D, D), lambda i: (0, 0, 0)),
            pl.BlockSpec((NR, 2, B, D), lambda i: (0, 0, i, 0)),
            pl.BlockSpec((NR, 2, B, D), lambda i: (0, 0, i, 0)),
        ],
        out_specs=[row_spec, row_spec, row_spec, row_spec],
        out_shape=[jax.ShapeDtypeStruct((N, D), jnp.float32)] * 4,
    )(*xs, W_roots, bs, W_rels, acc, cnt)


def kernel(x_author, W_root_author, b_root_author,
           x_field_of_study, W_root_field_of_study, b_root_field_of_study,
           x_institution, W_root_institution, b_root_institution,
           x_paper, W_root_paper, b_root_paper,
           W_rel_author_affiliated_with_institution, ei_author_affiliated_with_institution,
           W_rel_institution_to_author, ei_institution_to_author,
           W_rel_author_writes_paper, ei_author_writes_paper,
           W_rel_paper_to_author, ei_paper_to_author,
           W_rel_paper_cites_paper, ei_paper_cites_paper,
           W_rel_paper_has_topic_field_of_study, ei_paper_has_topic_field_of_study,
           W_rel_field_of_study_to_paper, ei_field_of_study_to_paper):
    xs = [x_author, x_field_of_study, x_institution, x_paper]
    eis = [ei_author_affiliated_with_institution, ei_institution_to_author,
           ei_author_writes_paper, ei_paper_to_author, ei_paper_cites_paper,
           ei_paper_has_topic_field_of_study, ei_field_of_study_to_paper]
    W_rels = [W_rel_author_affiliated_with_institution, W_rel_institution_to_author,
              W_rel_author_writes_paper, W_rel_paper_to_author, W_rel_paper_cites_paper,
              W_rel_paper_has_topic_field_of_study, W_rel_field_of_study_to_paper]

    x_stack = jnp.concatenate(xs, axis=0)

    npad = EPT_PAD - EPT
    pad_src = (jnp.arange(npad, dtype=jnp.int32) * 911) % (4 * N)
    pad_dst = N + (jnp.arange(npad, dtype=jnp.int32) % (ACC_ROWS - N))
    pad_src = jnp.broadcast_to(pad_src[None], (NW, npad))
    pad_dst = jnp.broadcast_to(pad_dst[None], (NW, npad))
    src_list, dst_list = [], []
    for r in range(NR):
        ei = eis[r]
        src = (ei[1] + REL_SRC_T[r] * N).astype(jnp.int32).reshape(NW, EPT)
        dst = ei[0].astype(jnp.int32).reshape(NW, EPT)
        src_list.append(jnp.concatenate([src, pad_src], axis=1).reshape(NW, NCH, CHUNK))
        dst_list.append(jnp.concatenate([dst, pad_dst], axis=1).reshape(NW, NCH, CHUNK))
    src_idx = jnp.stack(src_list)
    dst_idx = jnp.stack(dst_list)

    acc, cnt = _sc_segment_sums(x_stack, src_idx, dst_idx)

    W_roots = jnp.stack([W_root_author, W_root_field_of_study,
                         W_root_institution, W_root_paper])
    bs = jnp.stack([b_root_author, b_root_field_of_study,
                    b_root_institution, b_root_paper])
    W_rel_stack = jnp.stack(W_rels)

    outs = _tc_combine(xs, W_roots, bs, W_rel_stack, acc, cnt)
    return tuple(outs)
